# baseline (speedup 1.0000x reference)
import functools
import jax
import jax.numpy as jnp
from jax import lax
from jax.experimental import pallas as pl
from jax.experimental.pallas import tpu as pltpu
from jax.experimental.pallas import tpu_sc as plsc

N = 10000
E = 320000
D = 128
H = 128
B = 512
DE = 144
NPAD = 10240
NC = 2
NS = 16
NW = NC * NS
L = 16
CE = 128
CPT = 80
EPAD = NW * CPT * CE
STRIPE = NPAD // NS

_mesh = plsc.VectorSubcoreMesh(core_axis_name="c", subcore_axis_name="s")
_sc_params = pltpu.CompilerParams(needs_layout_passes=False,
                                  use_tc_tiling_on_sc=False)


def _i32(x):
    return x.astype(jnp.int32)


def _splat(ref, idx_splat):
    return plsc.load_gather(ref, [idx_splat])


def _full(i):
    return jnp.full((L,), i, dtype=jnp.int32)


@functools.partial(
    pl.kernel,
    out_type=(
        jax.ShapeDtypeStruct((B,), jnp.int32),
        jax.ShapeDtypeStruct((B,), jnp.int32),
        jax.ShapeDtypeStruct((B,), jnp.int32),
        jax.ShapeDtypeStruct((B,), jnp.int32),
    ),
    mesh=_mesh,
    compiler_params=_sc_params,
    scratch_types=[
        pltpu.VMEM((N,), jnp.int32),
        pltpu.VMEM((L,), jnp.int32),
        pltpu.VMEM((L,), jnp.int32),
        pltpu.VMEM((L,), jnp.int32),
        pltpu.VMEM((L,), jnp.int32),
    ],
)
def _seginfo_kernel(batch_hbm, starts_hbm, cnt_hbm, k1_hbm, k2_hbm,
                    batch_v, b_st, b_cnt, b_k1, b_k2):
    wid = lax.axis_index("s") * NC + lax.axis_index("c")
    pltpu.sync_copy(batch_hbm, batch_v)
    g = wid * L + lax.iota(jnp.int32, L)
    def lower_bound(tgt):
        def step(_, carry):
            lo, hi = carry
            mid = lax.shift_right_logical(lo + hi, 1)
            v = _splat(batch_v, mid)
            less = v < tgt
            return jnp.where(less, mid + 1, lo), jnp.where(less, hi, mid)
        lo0 = jnp.zeros((L,), jnp.int32)
        hi0 = jnp.full((L,), N, jnp.int32)
        lo, _ = lax.fori_loop(0, 14, step, (lo0, hi0))
        return lo

    st = lower_bound(g)
    en = lower_bound(g + 1)
    cnt = en - st
    k1 = (4 * cnt + 4) // 5
    k2 = (k1 + 1) // 2
    b_st[...] = st
    b_cnt[...] = cnt
    b_k1[...] = k1
    b_k2[...] = k2
    base = wid * L
    pltpu.sync_copy(b_st, starts_hbm.at[pl.ds(base, L)])
    pltpu.sync_copy(b_cnt, cnt_hbm.at[pl.ds(base, L)])
    pltpu.sync_copy(b_k1, k1_hbm.at[pl.ds(base, L)])
    pltpu.sync_copy(b_k2, k2_hbm.at[pl.ds(base, L)])


@functools.partial(
    pl.kernel,
    out_type=jax.ShapeDtypeStruct((NC, NPAD, DE), jnp.float32),
    mesh=_mesh,
    compiler_params=_sc_params,
    scratch_types=[
        pltpu.VMEM_SHARED((NPAD, DE), jnp.float32),
        pltpu.VMEM((CE, DE), jnp.float32),
        pltpu.VMEM((CE, DE), jnp.float32),
        pltpu.VMEM((CE,), jnp.int32),
        pltpu.VMEM((CE,), jnp.int32),
        pltpu.VMEM((CE,), jnp.int32),
        pltpu.VMEM((CE,), jnp.int32),
        pltpu.SemaphoreType.DMA,
        pltpu.SemaphoreType.DMA,
    ],
)
def _edge_agg_kernel(feat_hbm, src_hbm, dst_hbm, acc_hbm,
                     acc_sh, rows_a, rows_b, src_a, dst_a, src_b, dst_b,
                     sem_a, sem_b):
    cid = lax.axis_index("c")
    sid = lax.axis_index("s")
    wid = sid * NC + cid

    def zrow(r, _):
        def zcol(c, _):
            rows_a[r, pl.ds(c * L, L)] = jnp.zeros((L,), jnp.float32)
            return 0
        return lax.fori_loop(0, DE // L, zcol, 0)
    lax.fori_loop(0, CE, zrow, 0)
    stripe0 = sid * STRIPE
    for q in range(STRIPE // CE):
        pltpu.sync_copy(rows_a, acc_sh.at[pl.ds(stripe0 + q * CE, CE)])
    plsc.subcore_barrier()

    tile0 = wid * CPT

    def load_idx(c, sv, dv):
        base = (tile0 + c) * CE
        pltpu.sync_copy(src_hbm.at[pl.ds(base, CE)], sv)
        pltpu.sync_copy(dst_hbm.at[pl.ds(base, CE)], dv)

    def wait_gather(rows, sem):
        pltpu.make_async_copy(feat_hbm.at[pl.ds(0, CE)], rows, sem).wait()

    load_idx(0, src_a, dst_a)
    pltpu.async_copy(feat_hbm.at[src_a], rows_a, sem_a)

    def pair(t, _):
        o = 2 * t + 1
        load_idx(o, src_b, dst_b)
        pltpu.async_copy(feat_hbm.at[src_b], rows_b, sem_b)
        wait_gather(rows_a, sem_a)
        pltpu.sync_copy(rows_a, acc_sh.at[dst_a], add=True)
        nxt = jnp.minimum(2 * t + 2, CPT - 1)
        load_idx(nxt, src_a, dst_a)
        pltpu.async_copy(feat_hbm.at[src_a], rows_a, sem_a)
        wait_gather(rows_b, sem_b)
        pltpu.sync_copy(rows_b, acc_sh.at[dst_b], add=True)
        return 0
    lax.fori_loop(0, CPT // 2, pair, 0)
    wait_gather(rows_a, sem_a)
    plsc.subcore_barrier()

    for q in range(STRIPE // CE):
        r0 = stripe0 + q * CE
        pltpu.sync_copy(acc_sh.at[pl.ds(r0, CE)], rows_a)
        pltpu.sync_copy(rows_a, acc_hbm.at[cid].at[pl.ds(r0, CE)])


@functools.partial(
    pl.kernel,
    out_type=(
        jax.ShapeDtypeStruct((NPAD,), jnp.float32),
        jax.ShapeDtypeStruct((NPAD,), jnp.int32),
    ),
    mesh=_mesh,
    compiler_params=_sc_params,
    scratch_types=[
        pltpu.VMEM((NPAD,), jnp.float32),
        pltpu.VMEM((NPAD,), jnp.int32),
        pltpu.VMEM((NPAD,), jnp.float32),
        pltpu.VMEM((NPAD,), jnp.int32),
        pltpu.VMEM((B,), jnp.int32),
        pltpu.VMEM((B,), jnp.int32),
        pltpu.VMEM((B,), jnp.int32),
        pltpu.VMEM((NPAD // NW,), jnp.float32),
        pltpu.VMEM((NPAD // NW,), jnp.int32),
    ],
)
def _topk_kernel(score_hbm, tie_hbm, elig_hbm, batch_hbm, starts_hbm,
                 cnt_hbm, kk_hbm, keep_hbm, rank_hbm,
                 score_v, tie_v, elig_v, batch_v, starts_v, cnt_v, kk_v,
                 keep_o, rank_o):
    wid = lax.axis_index("s") * NC + lax.axis_index("c")
    pltpu.sync_copy(score_hbm, score_v)
    pltpu.sync_copy(tie_hbm, tie_v)
    pltpu.sync_copy(elig_hbm, elig_v)
    pltpu.sync_copy(batch_hbm, batch_v)
    pltpu.sync_copy(starts_hbm, starts_v)
    pltpu.sync_copy(cnt_hbm, cnt_v)
    pltpu.sync_copy(kk_hbm, kk_v)

    base = wid * (NPAD // NW)
    nloc = jnp.maximum(0, jnp.minimum(NPAD // NW, N - base))
    lane = lax.iota(jnp.int32, L)

    def node(il, _):
        i = base + il
        i_s = _full(i)
        g = _splat(batch_v, i_s)
        st = jnp.max(_splat(starts_v, g))
        cn = jnp.max(_splat(cnt_v, g))
        kg = jnp.max(_splat(kk_v, g))
        s_i = _splat(score_v, i_s)
        t_i = _splat(tie_v, i_s)
        e_i = jnp.max(_splat(elig_v, i_s))
        en = st + cn
        nch = (cn + (L - 1)) // L

        def chunk(t, acc):
            j = st + t * L
            sv = score_v[pl.ds(j, L)]
            tv = tie_v[pl.ds(j, L)]
            ev = elig_v[pl.ds(j, L)]
            jv = j + lane
            better = (sv > s_i) | ((sv == s_i) & (tv < t_i))
            m = (jv < en) & (ev > 0.0) & better
            return acc + jnp.sum(jnp.where(m, 1, 0))
        rank = lax.fori_loop(0, nch, chunk, jnp.int32(0))

        kept = (e_i > 0.0) & (rank < kg)
        kv = jnp.where(kept, jnp.float32(1.0), jnp.float32(0.0))
        il_s = _full(il)
        m0 = lane == 0
        plsc.store_scatter(keep_o, [il_s], jnp.full((L,), 1.0, jnp.float32) * kv,
                           mask=m0)
        plsc.store_scatter(rank_o, [il_s], jnp.zeros((L,), jnp.int32) + rank,
                           mask=m0)
        return 0
    lax.fori_loop(0, nloc, node, 0)

    pltpu.sync_copy(keep_o, keep_hbm.at[pl.ds(base, NPAD // NW)])
    pltpu.sync_copy(rank_o, rank_hbm.at[pl.ds(base, NPAD // NW)])


CP = 64


@functools.partial(
    pl.kernel,
    out_type=jax.ShapeDtypeStruct((B, 2 * H), jnp.float32),
    mesh=_mesh,
    compiler_params=_sc_params,
    scratch_types=[
        pltpu.VMEM((CP, DE), jnp.float32),
        pltpu.VMEM((B,), jnp.int32),
        pltpu.VMEM((B,), jnp.int32),
        pltpu.VMEM((B,), jnp.int32),
        pltpu.VMEM((L, 2 * H), jnp.float32),
        pltpu.SemaphoreType.DMA,
    ],
)
def _pool_kernel(feat_hbm, starts_hbm, cnt_hbm, kc_hbm, out_hbm,
                 rows_v, starts_v, cnt_v, kc_v, out_v, sem):
    wid = lax.axis_index("s") * NC + lax.axis_index("c")
    pltpu.sync_copy(starts_hbm, starts_v)
    pltpu.sync_copy(cnt_hbm, cnt_v)
    pltpu.sync_copy(kc_hbm, kc_v)
    g0 = wid * L
    NEG = jnp.float32(-3.0e38)

    for k in range(L):
        gk = _full(g0 + k)
        st = jnp.max(_splat(starts_v, gk))
        cn = jnp.max(_splat(cnt_v, gk))
        kc = jnp.max(_splat(kc_v, gk))
        for c in range(H // L):
            out_v[k, pl.ds(c * L, L)] = jnp.full((L,), NEG)
            out_v[k, pl.ds(H + c * L, L)] = jnp.zeros((L,), jnp.float32)
        nch = (cn + (CP - 1)) // CP

        def chunk(q, _):
            r0 = st + q * CP
            pltpu.async_copy(feat_hbm.at[pl.ds(r0, CP)], rows_v, sem).wait()
            nrow = jnp.minimum(CP, cn - q * CP)

            def row(r, _):
                kf = _splat(rows_v.at[r], _full(H))
                km = kf > 0.0
                for c in range(H // L):
                    v = rows_v[r, pl.ds(c * L, L)]
                    sacc = out_v[k, pl.ds(H + c * L, L)]
                    out_v[k, pl.ds(H + c * L, L)] = sacc + v
                    macc = out_v[k, pl.ds(c * L, L)]
                    out_v[k, pl.ds(c * L, L)] = jnp.where(
                        km, jnp.maximum(macc, v), macc)
                return 0
            lax.fori_loop(0, nrow, row, 0)
            return 0
        lax.fori_loop(0, nch, chunk, 0)

        any_kept = kc > 0
        for c in range(H // L):
            mv = out_v[k, pl.ds(c * L, L)]
            out_v[k, pl.ds(c * L, L)] = jnp.where(
                any_kept, mv, jnp.zeros((L,), jnp.float32))

    pltpu.sync_copy(out_v, out_hbm.at[pl.ds(g0, L)])


def _conv_dense_body(acc_ref, xext_ref, wl_ref, wr_ref, b_ref, p_ref,
                     pnorm_ref, h_ref, score_ref):
    acc = acc_ref[0] + acc_ref[1]
    s = acc[:N, :H]
    c = acc[:N, H:H + 1]
    agg = s / jnp.maximum(c, 1.0)
    xin = xext_ref[:N, :H]
    h = jnp.maximum(
        jnp.dot(agg, wl_ref[...])
        + jnp.dot(xin, wr_ref[...])
        + b_ref[...][None, :], 0.0)
    sc = jnp.tanh(jnp.dot(h, p_ref[...][:, None]) / pnorm_ref[0, 0])
    h_ref[...] = h
    score_ref[...] = jnp.concatenate(
        [sc, jnp.zeros((NPAD - N, 1), jnp.float32)], axis=0)


_conv_dense = pl.pallas_call(
    _conv_dense_body,
    out_shape=(
        jax.ShapeDtypeStruct((N, H), jnp.float32),
        jax.ShapeDtypeStruct((NPAD, 1), jnp.float32),
    ),
)


def _gate_body(h_ref, score_ref, keep_ref, out_ref):
    w = score_ref[:N] * keep_ref[:N]
    rows = h_ref[...] * w
    out_ref[...] = jnp.zeros((NPAD, DE), jnp.float32)
    out_ref[:N, :H] = rows
    out_ref[:N, H:H + 1] = keep_ref[:N]


_gate = pl.pallas_call(
    _gate_body,
    out_shape=jax.ShapeDtypeStruct((NPAD, DE), jnp.float32),
)


def _mlp_body(x1_ref, x2_ref, wp1_ref, bp1_ref, wp2_ref, bp2_ref,
              out_ref, lsm_ref):
    xo = x1_ref[...] + x2_ref[...]
    z = jnp.maximum(
        jnp.dot(xo, wp1_ref[...], preferred_element_type=jnp.float32)
        + bp1_ref[...][None, :], 0.0)
    o = (jnp.dot(z, wp2_ref[...], preferred_element_type=jnp.float32)
         + bp2_ref[...][None, :])
    m = jnp.max(o, axis=1, keepdims=True)
    lse = m + jnp.log(jnp.sum(jnp.exp(o - m), axis=1, keepdims=True))
    out_ref[...] = o
    lsm_ref[...] = o - lse


_mlp = pl.pallas_call(
    _mlp_body,
    out_shape=(
        jax.ShapeDtypeStruct((B, 2), jnp.float32),
        jax.ShapeDtypeStruct((B, 2), jnp.float32),
    ),
)


def kernel(x, edge_index, edge_attr, batch, W1l, W1r, b1, p1, W2l, W2r, b2,
           p2, Wp1, bp1, Wp2, bp2):
    del edge_attr
    batch = _i32(batch)
    src = jnp.pad(_i32(edge_index[0]), (0, EPAD - E))
    dst = jnp.pad(_i32(edge_index[1]), (0, EPAD - E), constant_values=N)

    starts, cnt, k1, k2 = _seginfo_kernel(batch)
    batch_p = jnp.pad(batch, (0, NPAD - N))
    x_ext = jnp.concatenate(
        [x, jnp.ones((N, 1), jnp.float32), jnp.zeros((N, DE - H - 1), jnp.float32)],
        axis=1)
    x_ext = jnp.pad(x_ext, ((0, NPAD - N), (0, 0)))
    acc1 = _edge_agg_kernel(x_ext, src, dst)
    h, score1 = _conv_dense(acc1, x_ext, W1l, W1r, b1, p1,
                             jnp.reshape(jnp.linalg.norm(p1), (1, 1)))
    s1flat = jnp.reshape(score1, (NPAD,))
    keep1, rank1 = _topk_kernel(
        s1flat, jnp.arange(NPAD, dtype=jnp.int32),
        jnp.ones((NPAD,), jnp.float32), batch_p, starts, cnt, k1)
    h1ext = _gate(h, score1, jnp.reshape(keep1, (NPAD, 1)))
    x1 = _pool_kernel(h1ext, starts, cnt, k1)

    acc2 = _edge_agg_kernel(h1ext, src, dst)
    h2, score2 = _conv_dense(acc2, h1ext, W2l, W2r, b2, p2,
                              jnp.reshape(jnp.linalg.norm(p2), (1, 1)))
    s2flat = jnp.reshape(score2, (NPAD,))
    keep2, _ = _topk_kernel(s2flat, rank1, keep1, batch_p, starts, cnt, k2)
    h2ext = _gate(h2, score2, jnp.reshape(keep2, (NPAD, 1)))
    x2 = _pool_kernel(h2ext, starts, cnt, k2)

    out, lsm = _mlp(x1, x2, Wp1, bp1, Wp2, bp2)
    return out, lsm


# speedup vs baseline: 1.1470x; 1.1470x over previous
import functools
import jax
import jax.numpy as jnp
from jax import lax
from jax.experimental import pallas as pl
from jax.experimental.pallas import tpu as pltpu
from jax.experimental.pallas import tpu_sc as plsc

N = 10000
E = 320000
D = 128
H = 128
B = 512
DE = 144
NPAD = 10240
NC = 2
NS = 16
NW = NC * NS
L = 16
CE = 128
CPT = 79
EPAD = NW * CPT * CE
STRIPE = NPAD // NS

_mesh = plsc.VectorSubcoreMesh(core_axis_name="c", subcore_axis_name="s")
_sc_params = pltpu.CompilerParams(needs_layout_passes=False,
                                  use_tc_tiling_on_sc=False)


def _i32(x):
    return x.astype(jnp.int32)


def _splat(ref, idx_splat):
    return plsc.load_gather(ref, [idx_splat])


def _full(i):
    return jnp.full((L,), i, dtype=jnp.int32)


@functools.partial(
    pl.kernel,
    out_type=(
        jax.ShapeDtypeStruct((B,), jnp.int32),
        jax.ShapeDtypeStruct((B,), jnp.int32),
        jax.ShapeDtypeStruct((B,), jnp.int32),
        jax.ShapeDtypeStruct((B,), jnp.int32),
    ),
    mesh=_mesh,
    compiler_params=_sc_params,
    scratch_types=[
        pltpu.VMEM((N,), jnp.int32),
        pltpu.VMEM((L,), jnp.int32),
        pltpu.VMEM((L,), jnp.int32),
        pltpu.VMEM((L,), jnp.int32),
        pltpu.VMEM((L,), jnp.int32),
    ],
)
def _seginfo_kernel(batch_hbm, starts_hbm, cnt_hbm, k1_hbm, k2_hbm,
                    batch_v, b_st, b_cnt, b_k1, b_k2):
    wid = lax.axis_index("s") * NC + lax.axis_index("c")
    pltpu.sync_copy(batch_hbm, batch_v)
    g = wid * L + lax.iota(jnp.int32, L)
    def lower_bound(tgt):
        def step(_, carry):
            lo, hi = carry
            mid = lax.shift_right_logical(lo + hi, 1)
            v = _splat(batch_v, mid)
            less = v < tgt
            return jnp.where(less, mid + 1, lo), jnp.where(less, hi, mid)
        lo0 = jnp.zeros((L,), jnp.int32)
        hi0 = jnp.full((L,), N, jnp.int32)
        lo, _ = lax.fori_loop(0, 14, step, (lo0, hi0))
        return lo

    st = lower_bound(g)
    en = lower_bound(g + 1)
    cnt = en - st
    k1 = (4 * cnt + 4) // 5
    k2 = (k1 + 1) // 2
    b_st[...] = st
    b_cnt[...] = cnt
    b_k1[...] = k1
    b_k2[...] = k2
    base = wid * L
    pltpu.sync_copy(b_st, starts_hbm.at[pl.ds(base, L)])
    pltpu.sync_copy(b_cnt, cnt_hbm.at[pl.ds(base, L)])
    pltpu.sync_copy(b_k1, k1_hbm.at[pl.ds(base, L)])
    pltpu.sync_copy(b_k2, k2_hbm.at[pl.ds(base, L)])


@functools.partial(
    pl.kernel,
    out_type=jax.ShapeDtypeStruct((NC, NPAD, DE), jnp.float32),
    mesh=_mesh,
    compiler_params=_sc_params,
    scratch_types=[
        pltpu.VMEM_SHARED((NPAD, DE), jnp.float32),
        pltpu.VMEM((CE, DE), jnp.float32),
        pltpu.VMEM((CE, DE), jnp.float32),
        pltpu.VMEM((CE,), jnp.int32),
        pltpu.VMEM((CE,), jnp.int32),
        pltpu.VMEM((CE,), jnp.int32),
        pltpu.VMEM((CE,), jnp.int32),
        pltpu.SemaphoreType.DMA,
        pltpu.SemaphoreType.DMA,
    ],
)
def _edge_agg_kernel(feat_hbm, src_hbm, dst_hbm, acc_hbm,
                     acc_sh, rows_a, rows_b, src_a, dst_a, src_b, dst_b,
                     sem_a, sem_b):
    cid = lax.axis_index("c")
    sid = lax.axis_index("s")
    wid = sid * NC + cid

    def zrow(r, _):
        def zcol(c, _):
            rows_a[r, pl.ds(c * L, L)] = jnp.zeros((L,), jnp.float32)
            return 0
        return lax.fori_loop(0, DE // L, zcol, 0)
    lax.fori_loop(0, CE, zrow, 0)
    stripe0 = sid * STRIPE
    for q in range(STRIPE // CE):
        pltpu.sync_copy(rows_a, acc_sh.at[pl.ds(stripe0 + q * CE, CE)])
    plsc.subcore_barrier()

    def chunk(c, _):
        base = (wid * CPT + c) * CE
        pltpu.sync_copy(src_hbm.at[pl.ds(base, CE)], src_a)
        pltpu.sync_copy(dst_hbm.at[pl.ds(base, CE)], dst_a)
        pltpu.async_copy(feat_hbm.at[src_a], rows_a, sem_a).wait()
        pltpu.sync_copy(rows_a, acc_sh.at[dst_a], add=True)
        return 0
    lax.fori_loop(0, CPT, chunk, 0)
    plsc.subcore_barrier()

    for q in range(STRIPE // CE):
        r0 = stripe0 + q * CE
        pltpu.sync_copy(acc_sh.at[pl.ds(r0, CE)], rows_a)
        pltpu.sync_copy(rows_a, acc_hbm.at[cid].at[pl.ds(r0, CE)])


@functools.partial(
    pl.kernel,
    out_type=(
        jax.ShapeDtypeStruct((NPAD,), jnp.float32),
        jax.ShapeDtypeStruct((NPAD,), jnp.int32),
    ),
    mesh=_mesh,
    compiler_params=_sc_params,
    scratch_types=[
        pltpu.VMEM((NPAD,), jnp.float32),
        pltpu.VMEM((NPAD,), jnp.int32),
        pltpu.VMEM((NPAD,), jnp.float32),
        pltpu.VMEM((NPAD,), jnp.int32),
        pltpu.VMEM((B,), jnp.int32),
        pltpu.VMEM((B,), jnp.int32),
        pltpu.VMEM((B,), jnp.int32),
        pltpu.VMEM((NPAD // NW,), jnp.float32),
        pltpu.VMEM((NPAD // NW,), jnp.int32),
    ],
)
def _topk_kernel(score_hbm, tie_hbm, elig_hbm, batch_hbm, starts_hbm,
                 cnt_hbm, kk_hbm, keep_hbm, rank_hbm,
                 score_v, tie_v, elig_v, batch_v, starts_v, cnt_v, kk_v,
                 keep_o, rank_o):
    wid = lax.axis_index("s") * NC + lax.axis_index("c")
    pltpu.sync_copy(score_hbm, score_v)
    pltpu.sync_copy(tie_hbm, tie_v)
    pltpu.sync_copy(elig_hbm, elig_v)
    pltpu.sync_copy(batch_hbm, batch_v)
    pltpu.sync_copy(starts_hbm, starts_v)
    pltpu.sync_copy(cnt_hbm, cnt_v)
    pltpu.sync_copy(kk_hbm, kk_v)

    base = wid * (NPAD // NW)
    nloc = jnp.maximum(0, jnp.minimum(NPAD // NW, N - base))
    lane = lax.iota(jnp.int32, L)

    def node(il, _):
        i = base + il
        i_s = _full(i)
        g = _splat(batch_v, i_s)
        st = jnp.max(_splat(starts_v, g))
        cn = jnp.max(_splat(cnt_v, g))
        kg = jnp.max(_splat(kk_v, g))
        s_i = _splat(score_v, i_s)
        t_i = _splat(tie_v, i_s)
        e_i = jnp.max(_splat(elig_v, i_s))
        en = st + cn
        nch = (cn + (L - 1)) // L

        def chunk(t, acc):
            j = st + t * L
            sv = score_v[pl.ds(j, L)]
            tv = tie_v[pl.ds(j, L)]
            ev = elig_v[pl.ds(j, L)]
            jv = j + lane
            better = (sv > s_i) | ((sv == s_i) & (tv < t_i))
            m = (jv < en) & (ev > 0.0) & better
            return acc + jnp.sum(jnp.where(m, 1, 0))
        rank = lax.fori_loop(0, nch, chunk, jnp.int32(0))

        kept = (e_i > 0.0) & (rank < kg)
        kv = jnp.where(kept, jnp.float32(1.0), jnp.float32(0.0))
        il_s = _full(il)
        m0 = lane == 0
        plsc.store_scatter(keep_o, [il_s], jnp.full((L,), 1.0, jnp.float32) * kv,
                           mask=m0)
        plsc.store_scatter(rank_o, [il_s], jnp.zeros((L,), jnp.int32) + rank,
                           mask=m0)
        return 0
    lax.fori_loop(0, nloc, node, 0)

    pltpu.sync_copy(keep_o, keep_hbm.at[pl.ds(base, NPAD // NW)])
    pltpu.sync_copy(rank_o, rank_hbm.at[pl.ds(base, NPAD // NW)])


CP = 64


@functools.partial(
    pl.kernel,
    out_type=jax.ShapeDtypeStruct((B, 2 * H), jnp.float32),
    mesh=_mesh,
    compiler_params=_sc_params,
    scratch_types=[
        pltpu.VMEM((CP, DE), jnp.float32),
        pltpu.VMEM((B,), jnp.int32),
        pltpu.VMEM((B,), jnp.int32),
        pltpu.VMEM((B,), jnp.int32),
        pltpu.VMEM((L, 2 * H), jnp.float32),
        pltpu.SemaphoreType.DMA,
    ],
)
def _pool_kernel(feat_hbm, starts_hbm, cnt_hbm, kc_hbm, out_hbm,
                 rows_v, starts_v, cnt_v, kc_v, out_v, sem):
    wid = lax.axis_index("s") * NC + lax.axis_index("c")
    pltpu.sync_copy(starts_hbm, starts_v)
    pltpu.sync_copy(cnt_hbm, cnt_v)
    pltpu.sync_copy(kc_hbm, kc_v)
    g0 = wid * L
    NEG = jnp.float32(-3.0e38)

    for k in range(L):
        gk = _full(g0 + k)
        st = jnp.max(_splat(starts_v, gk))
        cn = jnp.max(_splat(cnt_v, gk))
        kc = jnp.max(_splat(kc_v, gk))
        for c in range(H // L):
            out_v[k, pl.ds(c * L, L)] = jnp.full((L,), NEG)
            out_v[k, pl.ds(H + c * L, L)] = jnp.zeros((L,), jnp.float32)
        nch = (cn + (CP - 1)) // CP

        def chunk(q, _):
            r0 = st + q * CP
            pltpu.async_copy(feat_hbm.at[pl.ds(r0, CP)], rows_v, sem).wait()
            nrow = jnp.minimum(CP, cn - q * CP)

            def row(r, _):
                kf = _splat(rows_v.at[r], _full(H))
                km = kf > 0.0
                for c in range(H // L):
                    v = rows_v[r, pl.ds(c * L, L)]
                    sacc = out_v[k, pl.ds(H + c * L, L)]
                    out_v[k, pl.ds(H + c * L, L)] = sacc + v
                    macc = out_v[k, pl.ds(c * L, L)]
                    out_v[k, pl.ds(c * L, L)] = jnp.where(
                        km, jnp.maximum(macc, v), macc)
                return 0
            lax.fori_loop(0, nrow, row, 0)
            return 0
        lax.fori_loop(0, nch, chunk, 0)

        any_kept = kc > 0
        for c in range(H // L):
            mv = out_v[k, pl.ds(c * L, L)]
            out_v[k, pl.ds(c * L, L)] = jnp.where(
                any_kept, mv, jnp.zeros((L,), jnp.float32))

    pltpu.sync_copy(out_v, out_hbm.at[pl.ds(g0, L)])


def _conv_dense_body(acc_ref, xext_ref, wl_ref, wr_ref, b_ref, p_ref,
                     pnorm_ref, h_ref, score_ref):
    acc = acc_ref[0] + acc_ref[1]
    s = acc[:N, :H]
    c = acc[:N, H:H + 1]
    agg = s / jnp.maximum(c, 1.0)
    xin = xext_ref[:N, :H]
    h = jnp.maximum(
        jnp.dot(agg, wl_ref[...])
        + jnp.dot(xin, wr_ref[...])
        + b_ref[...][None, :], 0.0)
    sc = jnp.tanh(jnp.dot(h, p_ref[...][:, None]) / pnorm_ref[0, 0])
    h_ref[...] = h
    score_ref[...] = jnp.concatenate(
        [sc, jnp.zeros((NPAD - N, 1), jnp.float32)], axis=0)


_conv_dense = pl.pallas_call(
    _conv_dense_body,
    out_shape=(
        jax.ShapeDtypeStruct((N, H), jnp.float32),
        jax.ShapeDtypeStruct((NPAD, 1), jnp.float32),
    ),
)


def _gate_body(h_ref, score_ref, keep_ref, out_ref):
    w = score_ref[:N] * keep_ref[:N]
    rows = h_ref[...] * w
    out_ref[...] = jnp.zeros((NPAD, DE), jnp.float32)
    out_ref[:N, :H] = rows
    out_ref[:N, H:H + 1] = keep_ref[:N]


_gate = pl.pallas_call(
    _gate_body,
    out_shape=jax.ShapeDtypeStruct((NPAD, DE), jnp.float32),
)


def _mlp_body(x1_ref, x2_ref, wp1_ref, bp1_ref, wp2_ref, bp2_ref,
              out_ref, lsm_ref):
    xo = x1_ref[...] + x2_ref[...]
    z = jnp.maximum(
        jnp.dot(xo, wp1_ref[...], preferred_element_type=jnp.float32)
        + bp1_ref[...][None, :], 0.0)
    o = (jnp.dot(z, wp2_ref[...], preferred_element_type=jnp.float32)
         + bp2_ref[...][None, :])
    m = jnp.max(o, axis=1, keepdims=True)
    lse = m + jnp.log(jnp.sum(jnp.exp(o - m), axis=1, keepdims=True))
    out_ref[...] = o
    lsm_ref[...] = o - lse


_mlp = pl.pallas_call(
    _mlp_body,
    out_shape=(
        jax.ShapeDtypeStruct((B, 2), jnp.float32),
        jax.ShapeDtypeStruct((B, 2), jnp.float32),
    ),
)


def kernel(x, edge_index, edge_attr, batch, W1l, W1r, b1, p1, W2l, W2r, b2,
           p2, Wp1, bp1, Wp2, bp2):
    del edge_attr
    batch = _i32(batch)
    src = jnp.pad(_i32(edge_index[0]), (0, EPAD - E))
    dst = jnp.pad(_i32(edge_index[1]), (0, EPAD - E), constant_values=N)

    starts, cnt, k1, k2 = _seginfo_kernel(batch)
    batch_p = jnp.pad(batch, (0, NPAD - N))
    x_ext = jnp.concatenate(
        [x, jnp.ones((N, 1), jnp.float32), jnp.zeros((N, DE - H - 1), jnp.float32)],
        axis=1)
    x_ext = jnp.pad(x_ext, ((0, NPAD - N), (0, 0)))
    acc1 = _edge_agg_kernel(x_ext, src, dst)
    h, score1 = _conv_dense(acc1, x_ext, W1l, W1r, b1, p1,
                             jnp.reshape(jnp.linalg.norm(p1), (1, 1)))
    s1flat = jnp.reshape(score1, (NPAD,))
    keep1, rank1 = _topk_kernel(
        s1flat, jnp.arange(NPAD, dtype=jnp.int32),
        jnp.ones((NPAD,), jnp.float32), batch_p, starts, cnt, k1)
    h1ext = _gate(h, score1, jnp.reshape(keep1, (NPAD, 1)))
    x1 = _pool_kernel(h1ext, starts, cnt, k1)

    acc2 = _edge_agg_kernel(h1ext, src, dst)
    h2, score2 = _conv_dense(acc2, h1ext, W2l, W2r, b2, p2,
                              jnp.reshape(jnp.linalg.norm(p2), (1, 1)))
    s2flat = jnp.reshape(score2, (NPAD,))
    keep2, _ = _topk_kernel(s2flat, rank1, keep1, batch_p, starts, cnt, k2)
    h2ext = _gate(h2, score2, jnp.reshape(keep2, (NPAD, 1)))
    x2 = _pool_kernel(h2ext, starts, cnt, k2)

    out, lsm = _mlp(x1, x2, Wp1, bp1, Wp2, bp2)
    return out, lsm
